# Initial kernel scaffold; baseline (speedup 1.0000x reference)
import jax
import jax.numpy as jnp
from jax.experimental import pallas as pl


def kernel(x, edge_index, W1, b1, W2, b2, eps1, W3, b3, W4, b4, eps2):
    raise NotImplementedError("write your pallas kernel here")


# speedup vs baseline: 12.2113x; 12.2113x over previous
import functools

import jax
import jax.numpy as jnp
from jax import lax
from jax.experimental import pallas as pl
from jax.experimental.pallas import tpu as pltpu
from jax.experimental.pallas import tpu_sc as plsc

_NC = 2
_NS = 16
_NW = _NC * _NS
_CH = 128


def _segment_sum_sc(y, srcw, dstw, zeros, n, h, nch):
    np_rows = zeros.shape[0]
    acc_rows = np_rows
    rpz = np_rows // _NS

    @functools.partial(
        pl.kernel,
        out_type=jax.ShapeDtypeStruct((_NC, np_rows, h), jnp.float32),
        mesh=plsc.VectorSubcoreMesh(core_axis_name="c", subcore_axis_name="s"),
        scratch_types=[
            pltpu.VMEM((nch, _CH), jnp.int32),
            pltpu.VMEM((nch, _CH), jnp.int32),
            pltpu.VMEM((_CH, h), jnp.float32),
            pltpu.VMEM_SHARED((acc_rows, h), jnp.float32),
            pltpu.SemaphoreType.DMA,
        ],
        compiler_params=pltpu.CompilerParams(use_tc_tiling_on_sc=False),
    )
    def seg_sum(y_hbm, src_hbm, dst_hbm, z_hbm, out_hbm,
                src_v, dst_v, rows_v, acc_sh, sem):
        c = lax.axis_index("c")
        s = lax.axis_index("s")
        w = c * _NS + s
        pltpu.sync_copy(src_hbm.at[w], src_v)
        pltpu.sync_copy(dst_hbm.at[w], dst_v)
        pltpu.sync_copy(z_hbm.at[pl.ds(s * rpz, rpz)],
                        acc_sh.at[pl.ds(s * rpz, rpz)])
        plsc.subcore_barrier()

        def body(j, carry):
            pltpu.async_copy(y_hbm.at[src_v.at[j]], rows_v, sem).wait()
            pltpu.sync_copy(rows_v, acc_sh.at[dst_v.at[j]], add=True)
            return carry

        lax.fori_loop(0, nch, body, 0)
        plsc.subcore_barrier()
        pltpu.sync_copy(acc_sh.at[pl.ds(s * rpz, rpz)],
                        out_hbm.at[c, pl.ds(s * rpz, rpz)])

    return seg_sum(y, srcw, dstw, zeros)


def _matmul_tc(x, w, bn):
    n, d = x.shape
    h = w.shape[1]

    def body(x_ref, w_ref, o_ref):
        o_ref[...] = jnp.dot(x_ref[...], w_ref[...],
                             preferred_element_type=jnp.float32)

    return pl.pallas_call(
        body,
        grid=(n // bn,),
        in_specs=[pl.BlockSpec((bn, d), lambda i: (i, 0)),
                  pl.BlockSpec((d, h), lambda i: (0, 0))],
        out_specs=pl.BlockSpec((bn, h), lambda i: (i, 0)),
        out_shape=jax.ShapeDtypeStruct((n, h), jnp.float32),
    )(x, w)


def _gin_mid_tc(eps, y, p0, p1, b1, w2, b2, w3, bn):
    n, h = y.shape

    def body(e_ref, y_ref, p0_ref, p1_ref, b1_ref, w2_ref, b2_ref, w3_ref,
             o_ref):
        t = ((1.0 + e_ref[0, 0]) * y_ref[...] + p0_ref[...] + p1_ref[...]
             + b1_ref[...])
        u = jnp.dot(jnp.maximum(t, 0.0), w2_ref[...],
                    preferred_element_type=jnp.float32) + b2_ref[...]
        o_ref[...] = jnp.dot(jnp.maximum(u, 0.0), w3_ref[...],
                             preferred_element_type=jnp.float32)

    zero = lambda i: (0, 0)
    return pl.pallas_call(
        body,
        grid=(n // bn,),
        in_specs=[pl.BlockSpec((1, 1), zero),
                  pl.BlockSpec((bn, h), lambda i: (i, 0)),
                  pl.BlockSpec((bn, h), lambda i: (i, 0)),
                  pl.BlockSpec((bn, h), lambda i: (i, 0)),
                  pl.BlockSpec((1, h), zero),
                  pl.BlockSpec((h, h), zero),
                  pl.BlockSpec((1, h), zero),
                  pl.BlockSpec((h, h), zero)],
        out_specs=pl.BlockSpec((bn, h), lambda i: (i, 0)),
        out_shape=jax.ShapeDtypeStruct((n, h), jnp.float32),
    )(eps, y, p0, p1, b1, w2, b2, w3)


def _gin_out_tc(eps, y, p0, p1, b3, w4, b4, bn):
    n, h = y.shape

    def body(e_ref, y_ref, p0_ref, p1_ref, b3_ref, w4_ref, b4_ref, o_ref):
        t = ((1.0 + e_ref[0, 0]) * y_ref[...] + p0_ref[...] + p1_ref[...]
             + b3_ref[...])
        v = jnp.dot(jnp.maximum(t, 0.0), w4_ref[...],
                    preferred_element_type=jnp.float32) + b4_ref[...]
        m = jnp.max(v, axis=1, keepdims=True)
        lse = jnp.log(jnp.sum(jnp.exp(v - m), axis=1, keepdims=True)) + m
        o_ref[...] = v - lse

    zero = lambda i: (0, 0)
    return pl.pallas_call(
        body,
        grid=(n // bn,),
        in_specs=[pl.BlockSpec((1, 1), zero),
                  pl.BlockSpec((bn, h), lambda i: (i, 0)),
                  pl.BlockSpec((bn, h), lambda i: (i, 0)),
                  pl.BlockSpec((bn, h), lambda i: (i, 0)),
                  pl.BlockSpec((1, h), zero),
                  pl.BlockSpec((h, h), zero),
                  pl.BlockSpec((1, h), zero)],
        out_specs=pl.BlockSpec((bn, h), lambda i: (i, 0)),
        out_shape=jax.ShapeDtypeStruct((n, h), jnp.float32),
    )(eps, y, p0, p1, b3, w4, b4)


def kernel(x, edge_index, W1, b1, W2, b2, eps1, W3, b3, W4, b4, eps2):
    n, d = x.shape
    h = W1.shape[1]
    e = edge_index.shape[1]
    epw = e // _NW
    nch = -(-epw // _CH)
    epwp = nch * _CH
    pad = epwp - epw

    rpt = -(-(n // _NS) // 8) * 8
    np_rows = _NS * rpt

    src = edge_index[0].reshape(_NW, epw)
    dst = edge_index[1].reshape(_NW, epw)
    if pad:
        pad_src = jnp.broadcast_to(
            (jnp.arange(pad, dtype=jnp.int32) * 89) % n, (_NW, pad))
        pad_dst = jnp.broadcast_to(
            n + (jnp.arange(pad, dtype=jnp.int32) % (np_rows - n)), (_NW, pad))
        src = jnp.concatenate([src, pad_src], axis=1)
        dst = jnp.concatenate([dst, pad_dst], axis=1)
    srcw = src.reshape(_NW, nch, _CH)
    dstw = dst.reshape(_NW, nch, _CH)
    zeros = jnp.zeros((np_rows, h), jnp.float32)
    e1 = jnp.reshape(eps1, (1, 1)).astype(jnp.float32)
    e2 = jnp.reshape(eps2, (1, 1)).astype(jnp.float32)
    b1r = b1.reshape(1, h)
    b2r = b2.reshape(1, h)
    b3r = b3.reshape(1, h)
    b4r = b4.reshape(1, h)
    bn = 2000

    y1 = _matmul_tc(x, W1, bn)
    parts1 = _segment_sum_sc(y1, srcw, dstw, zeros, n, h, nch)
    y2 = _gin_mid_tc(e1, y1, parts1[0, :n], parts1[1, :n], b1r, W2, b2r, W3,
                     bn)
    parts2 = _segment_sum_sc(y2, srcw, dstw, zeros, n, h, nch)
    return _gin_out_tc(e2, y2, parts2[0, :n], parts2[1, :n], b3r, W4, b4r, bn)


# speedup vs baseline: 21.1464x; 1.7317x over previous
import functools

import jax
import jax.numpy as jnp
from jax import lax
from jax.experimental import pallas as pl
from jax.experimental.pallas import tpu as pltpu
from jax.experimental.pallas import tpu_sc as plsc

_NC = 2
_NS = 16
_NW = _NC * _NS
_CH = 128


def _segment_sum_sc(y, srcw, dstw, zeros, n, h, nch):
    np_rows = zeros.shape[0]
    acc_rows = np_rows
    rpz = np_rows // _NS

    grp = 4
    ngr = nch // grp

    @functools.partial(
        pl.kernel,
        out_type=jax.ShapeDtypeStruct((_NC, np_rows, h), jnp.float32),
        mesh=plsc.VectorSubcoreMesh(core_axis_name="c", subcore_axis_name="s"),
        scratch_types=[
            pltpu.VMEM((nch, _CH), jnp.int32),
            pltpu.VMEM((nch, _CH), jnp.int32),
            pltpu.VMEM((grp, _CH, h), jnp.float32),
            pltpu.VMEM((grp, _CH, h), jnp.float32),
            pltpu.VMEM_SHARED((acc_rows, h), jnp.float32),
            pltpu.SemaphoreType.DMA,
            pltpu.SemaphoreType.DMA,
        ],
        compiler_params=pltpu.CompilerParams(use_tc_tiling_on_sc=False),
    )
    def seg_sum(y_hbm, src_hbm, dst_hbm, z_hbm, out_hbm,
                src_v, dst_v, rows0_v, rows1_v, acc_sh, sem0, sem1):
        c = lax.axis_index("c")
        s = lax.axis_index("s")
        w = c * _NS + s
        pltpu.sync_copy(src_hbm.at[w], src_v)
        pltpu.sync_copy(dst_hbm.at[w], dst_v)
        pltpu.sync_copy(z_hbm.at[pl.ds(s * rpz, rpz)],
                        acc_sh.at[pl.ds(s * rpz, rpz)])
        plsc.subcore_barrier()

        def gathers(g, buf, sem):
            for k in range(grp):
                pltpu.async_copy(y_hbm.at[src_v.at[g * grp + k]],
                                 buf.at[k], sem)

        def drains(g, buf, sem):
            for k in range(grp):
                pltpu.make_async_copy(y_hbm.at[src_v.at[g * grp + k]],
                                      buf.at[k], sem).wait()

        def scatters(g, buf):
            for k in range(grp):
                pltpu.sync_copy(buf.at[k], acc_sh.at[dst_v.at[g * grp + k]],
                                add=True)

        gathers(0, rows0_v, sem0)

        def body(i, carry):
            g0 = 2 * i
            g1 = g0 + 1
            gathers(g1, rows1_v, sem1)
            drains(g0, rows0_v, sem0)
            scatters(g0, rows0_v)

            @pl.when(g1 + 1 < ngr)
            def _():
                gathers(g1 + 1, rows0_v, sem0)

            drains(g1, rows1_v, sem1)
            scatters(g1, rows1_v)
            return carry

        lax.fori_loop(0, ngr // 2, body, 0)
        plsc.subcore_barrier()
        pltpu.sync_copy(acc_sh.at[pl.ds(s * rpz, rpz)],
                        out_hbm.at[c, pl.ds(s * rpz, rpz)])

    return seg_sum(y, srcw, dstw, zeros)


def _matmul_tc(x, w, bn):
    n, d = x.shape
    h = w.shape[1]

    def body(x_ref, w_ref, o_ref):
        o_ref[...] = jnp.dot(x_ref[...], w_ref[...],
                             preferred_element_type=jnp.float32)

    return pl.pallas_call(
        body,
        grid=(n // bn,),
        in_specs=[pl.BlockSpec((bn, d), lambda i: (i, 0)),
                  pl.BlockSpec((d, h), lambda i: (0, 0))],
        out_specs=pl.BlockSpec((bn, h), lambda i: (i, 0)),
        out_shape=jax.ShapeDtypeStruct((n, h), jnp.float32),
    )(x, w)


def _gin_mid_tc(eps, y, parts, b1, w2, b2, w3, bn):
    n, h = y.shape

    def body(e_ref, y_ref, p_ref, b1_ref, w2_ref, b2_ref, w3_ref, o_ref):
        t = ((1.0 + e_ref[0, 0]) * y_ref[...] + p_ref[0] + p_ref[1]
             + b1_ref[...])
        u = jnp.dot(jnp.maximum(t, 0.0), w2_ref[...],
                    preferred_element_type=jnp.float32) + b2_ref[...]
        o_ref[...] = jnp.dot(jnp.maximum(u, 0.0), w3_ref[...],
                             preferred_element_type=jnp.float32)

    zero = lambda i: (0, 0)
    return pl.pallas_call(
        body,
        grid=(n // bn,),
        in_specs=[pl.BlockSpec((1, 1), zero),
                  pl.BlockSpec((bn, h), lambda i: (i, 0)),
                  pl.BlockSpec((2, bn, h), lambda i: (0, i, 0)),
                  pl.BlockSpec((1, h), zero),
                  pl.BlockSpec((h, h), zero),
                  pl.BlockSpec((1, h), zero),
                  pl.BlockSpec((h, h), zero)],
        out_specs=pl.BlockSpec((bn, h), lambda i: (i, 0)),
        out_shape=jax.ShapeDtypeStruct((n, h), jnp.float32),
    )(eps, y, parts, b1, w2, b2, w3)


def _gin_out_tc(eps, y, parts, b3, w4, b4, bn):
    n, h = y.shape

    def body(e_ref, y_ref, p_ref, b3_ref, w4_ref, b4_ref, o_ref):
        t = ((1.0 + e_ref[0, 0]) * y_ref[...] + p_ref[0] + p_ref[1]
             + b3_ref[...])
        v = jnp.dot(jnp.maximum(t, 0.0), w4_ref[...],
                    preferred_element_type=jnp.float32) + b4_ref[...]
        m = jnp.max(v, axis=1, keepdims=True)
        lse = jnp.log(jnp.sum(jnp.exp(v - m), axis=1, keepdims=True)) + m
        o_ref[...] = v - lse

    zero = lambda i: (0, 0)
    return pl.pallas_call(
        body,
        grid=(n // bn,),
        in_specs=[pl.BlockSpec((1, 1), zero),
                  pl.BlockSpec((bn, h), lambda i: (i, 0)),
                  pl.BlockSpec((2, bn, h), lambda i: (0, i, 0)),
                  pl.BlockSpec((1, h), zero),
                  pl.BlockSpec((h, h), zero),
                  pl.BlockSpec((1, h), zero)],
        out_specs=pl.BlockSpec((bn, h), lambda i: (i, 0)),
        out_shape=jax.ShapeDtypeStruct((n, h), jnp.float32),
    )(eps, y, parts, b3, w4, b4)


def kernel(x, edge_index, W1, b1, W2, b2, eps1, W3, b3, W4, b4, eps2):
    n, d = x.shape
    h = W1.shape[1]
    e = edge_index.shape[1]
    epw = e // _NW
    nch = -(-(-(-epw // _CH)) // 8) * 8
    epwp = nch * _CH
    pad = epwp - epw

    rpt = -(-(n // _NS) // 8) * 8
    np_rows = _NS * rpt

    src = edge_index[0].reshape(_NW, epw)
    dst = edge_index[1].reshape(_NW, epw)
    if pad:
        pad_src = jnp.broadcast_to(
            (jnp.arange(pad, dtype=jnp.int32) * 89) % n, (_NW, pad))
        pad_dst = jnp.broadcast_to(
            n + (jnp.arange(pad, dtype=jnp.int32) % (np_rows - n)), (_NW, pad))
        src = jnp.concatenate([src, pad_src], axis=1)
        dst = jnp.concatenate([dst, pad_dst], axis=1)
    srcw = src.reshape(_NW, nch, _CH)
    dstw = dst.reshape(_NW, nch, _CH)
    zeros = jnp.zeros((np_rows, h), jnp.float32)
    e1 = jnp.reshape(eps1, (1, 1)).astype(jnp.float32)
    e2 = jnp.reshape(eps2, (1, 1)).astype(jnp.float32)
    b1r = b1.reshape(1, h)
    b2r = b2.reshape(1, h)
    b3r = b3.reshape(1, h)
    b4r = b4.reshape(1, h)
    bn = 2000

    y1 = _matmul_tc(x, W1, bn)
    parts1 = _segment_sum_sc(y1, srcw, dstw, zeros, n, h, nch)
    y2 = _gin_mid_tc(e1, y1, parts1, b1r, W2, b2r, W3, bn)
    parts2 = _segment_sum_sc(y2, srcw, dstw, zeros, n, h, nch)
    return _gin_out_tc(e2, y2, parts2, b3r, W4, b4r, bn)


# speedup vs baseline: 21.4266x; 1.0132x over previous
import functools

import jax
import jax.numpy as jnp
from jax import lax
from jax.experimental import pallas as pl
from jax.experimental.pallas import tpu as pltpu
from jax.experimental.pallas import tpu_sc as plsc

_NC = 2
_NS = 16
_NW = _NC * _NS
_CH = 128


def _segment_sum_sc(table, srcm, dstm, init0, zeros, h, init_3d):
    np_rows = zeros.shape[0]
    rpz = np_rows // _NS
    ni = init0.shape[0]
    ipr = ni // _NS
    nrows = srcm.shape[0]
    main = nrows // _NW
    ntail = nrows - main * _NW

    grp = 3
    ngr = main // grp
    assert grp * ngr == main and ngr % 2 == 0
    assert ipr * _NS == ni and (ipr * h) % 8 == 0

    @functools.partial(
        pl.kernel,
        out_type=[jax.ShapeDtypeStruct((np_rows, h), jnp.float32),
                  jax.ShapeDtypeStruct((np_rows, h), jnp.float32)],
        mesh=plsc.VectorSubcoreMesh(core_axis_name="c", subcore_axis_name="s"),
        scratch_types=[
            pltpu.VMEM((main + 1, _CH), jnp.int32),
            pltpu.VMEM((main + 1, _CH), jnp.int32),
            pltpu.VMEM((grp, _CH, h), jnp.float32),
            pltpu.VMEM((grp, _CH, h), jnp.float32),
            pltpu.VMEM((_CH, h), jnp.float32),
            pltpu.VMEM_SHARED((np_rows, h), jnp.float32),
            pltpu.SemaphoreType.DMA,
            pltpu.SemaphoreType.DMA,
            pltpu.SemaphoreType.DMA,
        ],
        compiler_params=pltpu.CompilerParams(use_tc_tiling_on_sc=False),
    )
    def seg_sum(y_hbm, src_hbm, dst_hbm, i0_hbm, z_hbm, out0_hbm, out1_hbm,
                src_v, dst_v, rows0_v, rows1_v, tail_v, acc_sh,
                sem0, sem1, sem2):
        c = lax.axis_index("c")
        s = lax.axis_index("s")
        w = c * _NS + s
        pltpu.sync_copy(src_hbm.at[pl.ds(main * w, main)],
                        src_v.at[pl.ds(0, main)])
        pltpu.sync_copy(dst_hbm.at[pl.ds(main * w, main)],
                        dst_v.at[pl.ds(0, main)])

        @pl.when(w < ntail)
        def _():
            pltpu.sync_copy(src_hbm.at[pl.ds(main * _NW + w, 1)],
                            src_v.at[pl.ds(main, 1)])
            pltpu.sync_copy(dst_hbm.at[pl.ds(main * _NW + w, 1)],
                            dst_v.at[pl.ds(main, 1)])

        @pl.when(c == 0)
        def _():
            if init_3d:
                pltpu.sync_copy(i0_hbm.at[pl.ds(s * ipr, ipr), 1],
                                acc_sh.at[pl.ds(s * ipr, ipr)])
            else:
                pltpu.sync_copy(i0_hbm.at[pl.ds(s * ipr, ipr)],
                                acc_sh.at[pl.ds(s * ipr, ipr)])

        @pl.when(c == 1)
        def _():
            pltpu.sync_copy(z_hbm.at[pl.ds(s * rpz, rpz)],
                            acc_sh.at[pl.ds(s * rpz, rpz)])

        plsc.subcore_barrier()

        @pl.when(w < ntail)
        def _():
            pltpu.async_copy(y_hbm.at[src_v.at[main]], tail_v, sem2)

        def gathers(g, buf, sem):
            for k in range(grp):
                pltpu.async_copy(y_hbm.at[src_v.at[g * grp + k]],
                                 buf.at[k], sem)

        def drains(g, buf, sem):
            for k in range(grp):
                pltpu.make_async_copy(y_hbm.at[src_v.at[g * grp + k]],
                                      buf.at[k], sem).wait()

        def scatters(g, buf):
            for k in range(grp):
                pltpu.sync_copy(buf.at[k], acc_sh.at[dst_v.at[g * grp + k]],
                                add=True)

        gathers(0, rows0_v, sem0)

        def body(i, carry):
            g0 = 2 * i
            g1 = g0 + 1
            gathers(g1, rows1_v, sem1)
            drains(g0, rows0_v, sem0)
            scatters(g0, rows0_v)

            @pl.when(g1 + 1 < ngr)
            def _():
                gathers(g1 + 1, rows0_v, sem0)

            drains(g1, rows1_v, sem1)
            scatters(g1, rows1_v)
            return carry

        lax.fori_loop(0, ngr // 2, body, 0)

        @pl.when(w < ntail)
        def _():
            pltpu.make_async_copy(y_hbm.at[src_v.at[main]],
                                  tail_v, sem2).wait()
            pltpu.sync_copy(tail_v, acc_sh.at[dst_v.at[main]], add=True)

        plsc.subcore_barrier()
        @pl.when(c == 0)
        def _():
            pltpu.sync_copy(acc_sh.at[pl.ds(s * rpz, rpz)],
                            out0_hbm.at[pl.ds(s * rpz, rpz)])

        @pl.when(c == 1)
        def _():
            pltpu.sync_copy(acc_sh.at[pl.ds(s * rpz, rpz)],
                            out1_hbm.at[pl.ds(s * rpz, rpz)])

    return seg_sum(table, srcm, dstm, init0, zeros)


def _gin_in_tc(x, w1, eps):
    n, d = x.shape
    h = w1.shape[1]

    def body(x_ref, w_ref, e_ref, o_ref):
        y = jnp.dot(x_ref[...], w_ref[...],
                    preferred_element_type=jnp.float32)
        o_ref[...] = jnp.concatenate(
            [y, (1.0 + e_ref[0, 0]) * y, jnp.zeros((n, 2 * h), jnp.float32)],
            axis=1)

    return pl.pallas_call(
        body,
        out_shape=jax.ShapeDtypeStruct((n, 4 * h), jnp.float32),
    )(x, w1, eps)


def _gin_mid_tc(p0, p1, b1, w2, b2, w3, eps, nl, nlp):
    def body(p0_ref, p1_ref, b1_ref, w2_ref, b2_ref, w3_ref, e_ref,
             o1_ref, o2_ref):
        t = p0_ref[:nl] + p1_ref[:nl] + b1_ref[...]
        u = jnp.dot(jnp.maximum(t, 0.0), w2_ref[...],
                    preferred_element_type=jnp.float32) + b2_ref[...]
        y2 = jnp.dot(jnp.maximum(u, 0.0), w3_ref[...],
                     preferred_element_type=jnp.float32)
        o1_ref[pl.ds(0, nl)] = y2
        o2_ref[pl.ds(0, nl)] = (1.0 + e_ref[0, 0]) * y2

    return pl.pallas_call(
        body,
        out_shape=[jax.ShapeDtypeStruct((nlp, 128), jnp.float32),
                   jax.ShapeDtypeStruct((nlp, 128), jnp.float32)],
    )(p0, p1, b1, w2, b2, w3, eps)


def _gin_out_tc(q0, q1, b3, w4, b4, nl, nlp, h):
    def body(q0_ref, q1_ref, b3_ref, w4_ref, b4_ref, o_ref):
        t = q0_ref[:nl] + q1_ref[:nl] + b3_ref[...]
        v = jnp.dot(jnp.maximum(t, 0.0), w4_ref[...],
                    preferred_element_type=jnp.float32) + b4_ref[...]
        parts = []
        for j in range(4):
            vj = v[:, j * h:(j + 1) * h]
            mj = jnp.max(vj, axis=1, keepdims=True)
            sj = jnp.sum(jnp.exp(vj - mj), axis=1, keepdims=True)
            parts.append(vj - mj - jnp.log(sj))
        o_ref[pl.ds(0, nl)] = jnp.concatenate(parts, axis=1)

    return pl.pallas_call(
        body,
        out_shape=jax.ShapeDtypeStruct((nlp, 128), jnp.float32),
    )(q0, q1, b3, w4, b4)


def kernel(x, edge_index, W1, b1, W2, b2, eps1, W3, b3, W4, b4, eps2):
    n, d = x.shape
    h = W1.shape[1]
    e = edge_index.shape[1]
    nl = n * h // 128
    nlp = -(-nl // 8) * 8
    nrows = e // _CH

    rpt = -(-(n // _NS) // 8) * 8
    np_rows = _NS * rpt
    npl = np_rows * h // 128

    srcm = edge_index[0].reshape(nrows, _CH)
    dstm = edge_index[1].reshape(nrows, _CH)
    srcm4 = srcm * 4
    zeros = jnp.zeros((np_rows, h), jnp.float32)
    e1 = jnp.reshape(eps1, (1, 1)).astype(jnp.float32)
    e2 = jnp.reshape(eps2, (1, 1)).astype(jnp.float32)

    eye4 = jnp.eye(4, dtype=jnp.float32)
    W2b = jnp.kron(eye4, W2)
    W3b = jnp.kron(eye4, W3)
    W4b = jnp.kron(eye4, W4)
    b1t = jnp.tile(b1, 4).reshape(1, 128)
    b2t = jnp.tile(b2, 4).reshape(1, 128)
    b3t = jnp.tile(b3, 4).reshape(1, 128)
    b4t = jnp.tile(b4, 4).reshape(1, 128)

    o1 = _gin_in_tc(x, W1, e1)
    p0, p1 = _segment_sum_sc(o1.reshape(4 * n, h), srcm4, dstm,
                             o1.reshape(n, 4, h), zeros, h, True)
    y2, y2s = _gin_mid_tc(p0.reshape(npl, 128), p1.reshape(npl, 128),
                          b1t, W2b, b2t, W3b, e2, nl, nlp)
    q0, q1 = _segment_sum_sc(y2.reshape(nlp * 4, h), srcm, dstm,
                             y2s.reshape(nlp * 4, h), zeros, h, False)
    out = _gin_out_tc(q0.reshape(npl, 128), q1.reshape(npl, 128),
                      b3t, W4b, b4t, nl, nlp, h)
    return out.reshape(nlp * 4, h)[:n]


# speedup vs baseline: 26.4055x; 1.2324x over previous
import functools

import jax
import jax.numpy as jnp
from jax import lax
from jax.experimental import pallas as pl
from jax.experimental.pallas import tpu as pltpu
from jax.experimental.pallas import tpu_sc as plsc

_NC = 2
_NS = 16
_NW = _NC * _NS
_CH = 128


def _segment_sum_sc(table, edges, src4, init0, iidx, zeros, h):
    np_rows = zeros.shape[0]
    rpz = np_rows // _NS
    use_gather_init = iidx is not None
    if use_gather_init:
        nir = iidx.shape[0] // _NS
        ipr = nir * _CH
        assert ipr * _NS == np_rows
    else:
        ni = init0.shape[0]
        ipr = ni // _NS
        assert ipr * _NS == ni and (ipr * h) % 8 == 0
    nrows = edges.shape[0]
    main = nrows // _NW
    ntail = nrows - main * _NW

    grp = 3
    ngr = main // grp
    assert grp * ngr == main and ngr % 2 == 0

    in_types = [table, edges] + ([src4] if src4 is not None else []) \
        + ([iidx] if use_gather_init else [init0]) + [zeros]

    @functools.partial(
        pl.kernel,
        out_type=[jax.ShapeDtypeStruct((np_rows, h), jnp.float32),
                  jax.ShapeDtypeStruct((np_rows, h), jnp.float32)],
        mesh=plsc.VectorSubcoreMesh(core_axis_name="c", subcore_axis_name="s"),
        scratch_types=[
            pltpu.VMEM((main + 1, _CH), jnp.int32),
            pltpu.VMEM((main + 1, _CH), jnp.int32),
            pltpu.VMEM((grp, _CH, h), jnp.float32),
            pltpu.VMEM((grp, _CH, h), jnp.float32),
            pltpu.VMEM((_CH, h), jnp.float32),
            pltpu.VMEM((8, _CH), jnp.int32),
            pltpu.VMEM_SHARED((np_rows, h), jnp.float32),
            pltpu.SemaphoreType.DMA,
            pltpu.SemaphoreType.DMA,
            pltpu.SemaphoreType.DMA,
        ],
        compiler_params=pltpu.CompilerParams(use_tc_tiling_on_sc=False),
    )
    def seg_sum(*refs):
        it = iter(refs)
        y_hbm = next(it)
        e_hbm = next(it)
        s4_hbm = next(it) if src4 is not None else None
        if use_gather_init:
            i0_hbm, ii_hbm = None, next(it)
        else:
            i0_hbm, ii_hbm = next(it), None
        z_hbm = next(it)
        out0_hbm = next(it)
        out1_hbm = next(it)
        (src_v, dst_v, rows0_v, rows1_v, tail_v, iidx_v, acc_sh,
         sem0, sem1, sem2) = list(it)

        c = lax.axis_index("c")
        s = lax.axis_index("s")
        w = c * _NS + s
        if s4_hbm is not None:
            pltpu.sync_copy(s4_hbm.at[pl.ds(main * w, main)],
                            src_v.at[pl.ds(0, main)])
        else:
            pltpu.sync_copy(e_hbm.at[pl.ds(main * w, main), 0],
                            src_v.at[pl.ds(0, main)])
        pltpu.sync_copy(e_hbm.at[pl.ds(main * w, main), 1],
                        dst_v.at[pl.ds(0, main)])

        @pl.when(w < ntail)
        def _():
            if s4_hbm is not None:
                pltpu.sync_copy(s4_hbm.at[pl.ds(main * _NW + w, 1)],
                                src_v.at[pl.ds(main, 1)])
            else:
                pltpu.sync_copy(e_hbm.at[pl.ds(main * _NW + w, 1), 0],
                                src_v.at[pl.ds(main, 1)])
            pltpu.sync_copy(e_hbm.at[pl.ds(main * _NW + w, 1), 1],
                            dst_v.at[pl.ds(main, 1)])

        @pl.when(c == 0)
        def _():
            if use_gather_init:
                pltpu.sync_copy(ii_hbm.at[pl.ds(s * nir, nir)],
                                iidx_v.at[pl.ds(0, nir)])
                for k in range(nir):
                    pltpu.async_copy(y_hbm.at[iidx_v.at[k]],
                                     rows0_v.at[k % grp], sem0)
                    if k % grp == grp - 1 or k == nir - 1:
                        for k2 in range(k - k % grp, k + 1):
                            pltpu.make_async_copy(
                                y_hbm.at[iidx_v.at[k2]],
                                rows0_v.at[k2 % grp], sem0).wait()
                            pltpu.sync_copy(
                                rows0_v.at[k2 % grp],
                                acc_sh.at[pl.ds(s * ipr + k2 * _CH, _CH)])
            else:
                pltpu.sync_copy(i0_hbm.at[pl.ds(s * ipr, ipr)],
                                acc_sh.at[pl.ds(s * ipr, ipr)])

        @pl.when(c == 1)
        def _():
            pltpu.sync_copy(z_hbm.at[pl.ds(s * rpz, rpz)],
                            acc_sh.at[pl.ds(s * rpz, rpz)])

        plsc.subcore_barrier()

        @pl.when(w < ntail)
        def _():
            pltpu.async_copy(y_hbm.at[src_v.at[main]], tail_v, sem2)

        def gathers(g, buf, sem):
            for k in range(grp):
                pltpu.async_copy(y_hbm.at[src_v.at[g * grp + k]],
                                 buf.at[k], sem)

        def drains(g, buf, sem):
            for k in range(grp):
                pltpu.make_async_copy(y_hbm.at[src_v.at[g * grp + k]],
                                      buf.at[k], sem).wait()

        def scatters(g, buf):
            for k in range(grp):
                pltpu.sync_copy(buf.at[k], acc_sh.at[dst_v.at[g * grp + k]],
                                add=True)

        gathers(0, rows0_v, sem0)

        def body(i, carry):
            g0 = 2 * i
            g1 = g0 + 1
            gathers(g1, rows1_v, sem1)
            drains(g0, rows0_v, sem0)
            scatters(g0, rows0_v)

            @pl.when(g1 + 1 < ngr)
            def _():
                gathers(g1 + 1, rows0_v, sem0)

            drains(g1, rows1_v, sem1)
            scatters(g1, rows1_v)
            return carry

        lax.fori_loop(0, ngr // 2, body, 0)

        @pl.when(w < ntail)
        def _():
            pltpu.make_async_copy(y_hbm.at[src_v.at[main]],
                                  tail_v, sem2).wait()
            pltpu.sync_copy(tail_v, acc_sh.at[dst_v.at[main]], add=True)

        plsc.subcore_barrier()
        @pl.when(c == 0)
        def _():
            pltpu.sync_copy(acc_sh.at[pl.ds(s * rpz, rpz)],
                            out0_hbm.at[pl.ds(s * rpz, rpz)])

        @pl.when(c == 1)
        def _():
            pltpu.sync_copy(acc_sh.at[pl.ds(s * rpz, rpz)],
                            out1_hbm.at[pl.ds(s * rpz, rpz)])

    return seg_sum(*in_types)


def _gin_in_tc(x, w1, eps):
    n, d = x.shape
    h = w1.shape[1]

    def body(x_ref, w_ref, e_ref, o_ref):
        y = jnp.dot(x_ref[...], w_ref[...],
                    preferred_element_type=jnp.float32)
        o_ref[...] = jnp.concatenate(
            [y, (1.0 + e_ref[0, 0]) * y, jnp.zeros((n, 2 * h), jnp.float32)],
            axis=1)

    return pl.pallas_call(
        body,
        out_shape=jax.ShapeDtypeStruct((n, 4 * h), jnp.float32),
    )(x, w1, eps)


def _gin_mid_tc(p0, p1, b1, w2, b2, w3, eps, nl, nlp):
    def body(p0_ref, p1_ref, b1_ref, w2_ref, b2_ref, w3_ref, e_ref,
             o1_ref, o2_ref):
        t = p0_ref[:nl] + p1_ref[:nl] + b1_ref[...]
        u = jnp.dot(jnp.maximum(t, 0.0), w2_ref[...],
                    preferred_element_type=jnp.float32) + b2_ref[...]
        y2 = jnp.dot(jnp.maximum(u, 0.0), w3_ref[...],
                     preferred_element_type=jnp.float32)
        o1_ref[pl.ds(0, nl)] = y2
        o2_ref[pl.ds(0, nl)] = (1.0 + e_ref[0, 0]) * y2

    return pl.pallas_call(
        body,
        out_shape=[jax.ShapeDtypeStruct((nlp, 128), jnp.float32),
                   jax.ShapeDtypeStruct((nlp, 128), jnp.float32)],
    )(p0, p1, b1, w2, b2, w3, eps)


def _gin_out_tc(q0, q1, b3, w4, b4, nl, nlp, h):
    def body(q0_ref, q1_ref, b3_ref, w4_ref, b4_ref, o_ref):
        t = q0_ref[:nl] + q1_ref[:nl] + b3_ref[...]
        v = jnp.dot(jnp.maximum(t, 0.0), w4_ref[...],
                    preferred_element_type=jnp.float32) + b4_ref[...]
        parts = []
        for j in range(4):
            vj = v[:, j * h:(j + 1) * h]
            mj = jnp.max(vj, axis=1, keepdims=True)
            sj = jnp.sum(jnp.exp(vj - mj), axis=1, keepdims=True)
            parts.append(vj - mj - jnp.log(sj))
        o_ref[pl.ds(0, nl)] = jnp.concatenate(parts, axis=1)

    return pl.pallas_call(
        body,
        out_shape=jax.ShapeDtypeStruct((nlp, 128), jnp.float32),
    )(q0, q1, b3, w4, b4)


def kernel(x, edge_index, W1, b1, W2, b2, eps1, W3, b3, W4, b4, eps2):
    n, d = x.shape
    h = W1.shape[1]
    e = edge_index.shape[1]
    nl = n * h // 128
    nlp = -(-nl // 8) * 8
    nrows = e // _CH

    np_rows = -(-n // (_NS * _CH)) * _NS * _CH
    npl = np_rows * h // 128

    edges = jnp.transpose(edge_index.reshape(2, nrows, _CH), (1, 0, 2))
    src4 = edges[:, 0] * 4
    iidx = jnp.minimum(
        4 * jnp.arange(np_rows, dtype=jnp.int32) + 1, 4 * n - 3
    ).reshape(np_rows // _CH, _CH)
    zeros = jnp.zeros((np_rows, h), jnp.float32)
    e1 = jnp.reshape(eps1, (1, 1)).astype(jnp.float32)
    e2 = jnp.reshape(eps2, (1, 1)).astype(jnp.float32)

    eye4 = jnp.eye(4, dtype=jnp.float32)
    W2b = jnp.kron(eye4, W2)
    W3b = jnp.kron(eye4, W3)
    W4b = jnp.kron(eye4, W4)
    b1t = jnp.tile(b1, 4).reshape(1, 128)
    b2t = jnp.tile(b2, 4).reshape(1, 128)
    b3t = jnp.tile(b3, 4).reshape(1, 128)
    b4t = jnp.tile(b4, 4).reshape(1, 128)

    o1 = _gin_in_tc(x, W1, e1)
    p0, p1 = _segment_sum_sc(o1.reshape(4 * n, h), edges, src4,
                             None, iidx, zeros, h)
    y2, y2s = _gin_mid_tc(p0.reshape(npl, 128), p1.reshape(npl, 128),
                          b1t, W2b, b2t, W3b, e2, nl, nlp)
    q0, q1 = _segment_sum_sc(y2.reshape(nlp * 4, h), edges, None,
                             y2s.reshape(nlp * 4, h), None, zeros, h)
    out = _gin_out_tc(q0.reshape(npl, 128), q1.reshape(npl, 128),
                      b3t, W4b, b4t, nl, nlp, h)
    return out.reshape(nlp * 4, h)[:n]


# speedup vs baseline: 29.5698x; 1.1198x over previous
import functools

import jax
import jax.numpy as jnp
from jax import lax
from jax.experimental import pallas as pl
from jax.experimental.pallas import tpu as pltpu
from jax.experimental.pallas import tpu_sc as plsc

_NC = 2
_NS = 16
_NW = _NC * _NS
_CH = 128


def _segment_sum_sc(table, edges, scale_src, init0, iidx, zeros, h):
    np_rows = zeros.shape[0]
    rpz = np_rows // _NS
    use_gather_init = iidx is not None
    if use_gather_init:
        nir = iidx.shape[0] // _NS
        ipr = nir * _CH
        assert ipr * _NS == np_rows
    else:
        ni = init0.shape[0]
        ipr = ni // _NS
        assert ipr * _NS == ni and (ipr * h) % 8 == 0
    nrows = edges.shape[0]
    main = nrows // _NW
    ntail = nrows - main * _NW

    grp = 3
    ngr = main // grp
    assert grp * ngr == main and ngr % 2 == 0

    in_types = [table, edges] \
        + ([iidx] if use_gather_init else [init0]) + [zeros]

    @functools.partial(
        pl.kernel,
        out_type=[jax.ShapeDtypeStruct((np_rows, h), jnp.float32),
                  jax.ShapeDtypeStruct((np_rows, h), jnp.float32)],
        mesh=plsc.VectorSubcoreMesh(core_axis_name="c", subcore_axis_name="s"),
        scratch_types=[
            pltpu.VMEM((main + 1, 2, _CH), jnp.int32),
            pltpu.VMEM((grp, _CH, h), jnp.float32),
            pltpu.VMEM((grp, _CH, h), jnp.float32),
            pltpu.VMEM((_CH, h), jnp.float32),
            pltpu.VMEM((8, _CH), jnp.int32),
            pltpu.VMEM_SHARED((np_rows, h), jnp.float32),
            pltpu.SemaphoreType.DMA,
            pltpu.SemaphoreType.DMA,
            pltpu.SemaphoreType.DMA,
        ],
        compiler_params=pltpu.CompilerParams(use_tc_tiling_on_sc=False),
    )
    def seg_sum(*refs):
        it = iter(refs)
        y_hbm = next(it)
        e_hbm = next(it)
        if use_gather_init:
            i0_hbm, ii_hbm = None, next(it)
        else:
            i0_hbm, ii_hbm = next(it), None
        z_hbm = next(it)
        out0_hbm = next(it)
        out1_hbm = next(it)
        (ev, rows0_v, rows1_v, tail_v, iidx_v, acc_sh,
         sem0, sem1, sem2) = list(it)

        c = lax.axis_index("c")
        s = lax.axis_index("s")
        w = c * _NS + s
        pltpu.sync_copy(e_hbm.at[pl.ds(main * w, main)],
                        ev.at[pl.ds(0, main)])

        @pl.when(w < ntail)
        def _():
            pltpu.sync_copy(e_hbm.at[pl.ds(main * _NW + w, 1)],
                            ev.at[pl.ds(main, 1)])

        if scale_src:
            for i in range(main):
                for j in range(_CH // 16):
                    sl = (i, 0, pl.ds(16 * j, 16))
                    ev[sl] = ev[sl] * 4

            @pl.when(w < ntail)
            def _():
                for j in range(_CH // 16):
                    sl = (main, 0, pl.ds(16 * j, 16))
                    ev[sl] = ev[sl] * 4

        def src_at(ch):
            return ev.at[ch, 0]

        def dst_at(ch):
            return ev.at[ch, 1]

        @pl.when(c == 0)
        def _():
            if use_gather_init:
                pltpu.sync_copy(ii_hbm.at[pl.ds(s * nir, nir)],
                                iidx_v.at[pl.ds(0, nir)])
                for k in range(nir):
                    pltpu.async_copy(y_hbm.at[iidx_v.at[k]],
                                     rows0_v.at[k % grp], sem0)
                    if k % grp == grp - 1 or k == nir - 1:
                        for k2 in range(k - k % grp, k + 1):
                            pltpu.make_async_copy(
                                y_hbm.at[iidx_v.at[k2]],
                                rows0_v.at[k2 % grp], sem0).wait()
                            pltpu.sync_copy(
                                rows0_v.at[k2 % grp],
                                acc_sh.at[pl.ds(s * ipr + k2 * _CH, _CH)])
            else:
                pltpu.sync_copy(i0_hbm.at[pl.ds(s * ipr, ipr)],
                                acc_sh.at[pl.ds(s * ipr, ipr)])

        @pl.when(c == 1)
        def _():
            pltpu.sync_copy(z_hbm.at[pl.ds(s * rpz, rpz)],
                            acc_sh.at[pl.ds(s * rpz, rpz)])

        plsc.subcore_barrier()

        @pl.when(w < ntail)
        def _():
            pltpu.async_copy(y_hbm.at[src_at(main)], tail_v, sem2)

        def gathers(g, buf, sem):
            for k in range(grp):
                pltpu.async_copy(y_hbm.at[src_at(g * grp + k)],
                                 buf.at[k], sem)

        def drains(g, buf, sem):
            for k in range(grp):
                pltpu.make_async_copy(y_hbm.at[src_at(g * grp + k)],
                                      buf.at[k], sem).wait()

        def scatters(g, buf):
            for k in range(grp):
                pltpu.sync_copy(buf.at[k], acc_sh.at[dst_at(g * grp + k)],
                                add=True)

        gathers(0, rows0_v, sem0)

        def body(i, carry):
            g0 = 2 * i
            g1 = g0 + 1
            gathers(g1, rows1_v, sem1)
            drains(g0, rows0_v, sem0)
            scatters(g0, rows0_v)

            @pl.when(g1 + 1 < ngr)
            def _():
                gathers(g1 + 1, rows0_v, sem0)

            drains(g1, rows1_v, sem1)
            scatters(g1, rows1_v)
            return carry

        lax.fori_loop(0, ngr // 2, body, 0)

        @pl.when(w < ntail)
        def _():
            pltpu.make_async_copy(y_hbm.at[src_at(main)],
                                  tail_v, sem2).wait()
            pltpu.sync_copy(tail_v, acc_sh.at[dst_at(main)], add=True)

        plsc.subcore_barrier()
        @pl.when(c == 0)
        def _():
            pltpu.sync_copy(acc_sh.at[pl.ds(s * rpz, rpz)],
                            out0_hbm.at[pl.ds(s * rpz, rpz)])

        @pl.when(c == 1)
        def _():
            pltpu.sync_copy(acc_sh.at[pl.ds(s * rpz, rpz)],
                            out1_hbm.at[pl.ds(s * rpz, rpz)])

    return seg_sum(*in_types)


def _gin_in_tc(x, w1, eps):
    n, d = x.shape
    h = w1.shape[1]

    def body(x_ref, w_ref, e_ref, o_ref):
        y = jnp.dot(x_ref[...], w_ref[...],
                    preferred_element_type=jnp.float32)
        o_ref[...] = jnp.concatenate(
            [y, (1.0 + e_ref[0, 0]) * y, jnp.zeros((n, 2 * h), jnp.float32)],
            axis=1)

    return pl.pallas_call(
        body,
        out_shape=jax.ShapeDtypeStruct((n, 4 * h), jnp.float32),
    )(x, w1, eps)


def _gin_mid_tc(p0, p1, b1, w2, b2, w3, eps, nl, nlp):
    def body(p0_ref, p1_ref, b1_ref, w2_ref, b2_ref, w3_ref, e_ref,
             o1_ref, o2_ref):
        t = p0_ref[:nl] + p1_ref[:nl] + b1_ref[...]
        u = jnp.dot(jnp.maximum(t, 0.0), w2_ref[...],
                    preferred_element_type=jnp.float32) + b2_ref[...]
        y2 = jnp.dot(jnp.maximum(u, 0.0), w3_ref[...],
                     preferred_element_type=jnp.float32)
        o1_ref[pl.ds(0, nl)] = y2
        o2_ref[pl.ds(0, nl)] = (1.0 + e_ref[0, 0]) * y2

    return pl.pallas_call(
        body,
        out_shape=[jax.ShapeDtypeStruct((nlp, 128), jnp.float32),
                   jax.ShapeDtypeStruct((nlp, 128), jnp.float32)],
    )(p0, p1, b1, w2, b2, w3, eps)


def _gin_out_tc(q0, q1, b3, w4, b4, nl, nlp, h):
    def body(q0_ref, q1_ref, b3_ref, w4_ref, b4_ref, o_ref):
        t = q0_ref[:nl] + q1_ref[:nl] + b3_ref[...]
        v = jnp.dot(jnp.maximum(t, 0.0), w4_ref[...],
                    preferred_element_type=jnp.float32) + b4_ref[...]
        parts = []
        for j in range(4):
            vj = v[:, j * h:(j + 1) * h]
            mj = jnp.max(vj, axis=1, keepdims=True)
            sj = jnp.sum(jnp.exp(vj - mj), axis=1, keepdims=True)
            parts.append(vj - mj - jnp.log(sj))
        o_ref[pl.ds(0, nl)] = jnp.concatenate(parts, axis=1)

    return pl.pallas_call(
        body,
        out_shape=jax.ShapeDtypeStruct((nlp, 128), jnp.float32),
    )(q0, q1, b3, w4, b4)


def kernel(x, edge_index, W1, b1, W2, b2, eps1, W3, b3, W4, b4, eps2):
    n, d = x.shape
    h = W1.shape[1]
    e = edge_index.shape[1]
    nl = n * h // 128
    nlp = -(-nl // 8) * 8
    nrows = e // _CH

    np_rows = -(-n // (_NS * _CH)) * _NS * _CH
    npl = np_rows * h // 128

    edges = jnp.transpose(edge_index.reshape(2, nrows, _CH), (1, 0, 2))
    iidx = jnp.minimum(
        4 * jnp.arange(np_rows, dtype=jnp.int32) + 1, 4 * n - 3
    ).reshape(np_rows // _CH, _CH)
    zeros = jnp.zeros((np_rows, h), jnp.float32)
    e1 = jnp.reshape(eps1, (1, 1)).astype(jnp.float32)
    e2 = jnp.reshape(eps2, (1, 1)).astype(jnp.float32)

    eye4 = jnp.eye(4, dtype=jnp.float32)
    W2b = jnp.kron(eye4, W2)
    W3b = jnp.kron(eye4, W3)
    W4b = jnp.kron(eye4, W4)
    b1t = jnp.tile(b1, 4).reshape(1, 128)
    b2t = jnp.tile(b2, 4).reshape(1, 128)
    b3t = jnp.tile(b3, 4).reshape(1, 128)
    b4t = jnp.tile(b4, 4).reshape(1, 128)

    o1 = _gin_in_tc(x, W1, e1)
    p0, p1 = _segment_sum_sc(o1.reshape(4 * n, h), edges, True,
                             None, iidx, zeros, h)
    y2, y2s = _gin_mid_tc(p0.reshape(npl, 128), p1.reshape(npl, 128),
                          b1t, W2b, b2t, W3b, e2, nl, nlp)
    q0, q1 = _segment_sum_sc(y2.reshape(nlp * 4, h), edges, False,
                             y2s.reshape(nlp * 4, h), None, zeros, h)
    out = _gin_out_tc(q0.reshape(npl, 128), q1.reshape(npl, 128),
                      b3t, W4b, b4t, nl, nlp, h)
    return out.reshape(nlp * 4, h)[:n]


# speedup vs baseline: 30.2894x; 1.0243x over previous
import functools

import jax
import jax.numpy as jnp
from jax import lax
from jax.experimental import pallas as pl
from jax.experimental.pallas import tpu as pltpu
from jax.experimental.pallas import tpu_sc as plsc

_NC = 2
_NS = 16
_NW = _NC * _NS
_CH = 128


def _segment_sum_sc(table, edges, scale_src, init0, iidx, zeros, h):
    np_rows = zeros.shape[0]
    rpz = np_rows // _NS
    use_gather_init = iidx is not None
    if use_gather_init:
        nir = iidx.shape[0] // _NS
        ipr = nir * _CH
        assert ipr * _NS == np_rows
    else:
        ni = init0.shape[0]
        ipr = ni // _NS
        assert ipr * _NS == ni and (ipr * h) % 8 == 0
    nrows = edges.shape[0]
    main = nrows // _NW
    ntail = nrows - main * _NW

    grp = 3
    ngr = main // grp
    assert grp * ngr == main and ngr % 2 == 0

    in_types = [table, edges] \
        + ([iidx] if use_gather_init else [init0]) + [zeros]

    @functools.partial(
        pl.kernel,
        out_type=[jax.ShapeDtypeStruct((np_rows, h), jnp.float32),
                  jax.ShapeDtypeStruct((np_rows, h), jnp.float32)],
        mesh=plsc.VectorSubcoreMesh(core_axis_name="c", subcore_axis_name="s"),
        scratch_types=[
            pltpu.VMEM((main + 1, 2, _CH), jnp.int32),
            pltpu.VMEM((grp, _CH, h), jnp.float32),
            pltpu.VMEM((grp, _CH, h), jnp.float32),
            pltpu.VMEM((_CH, h), jnp.float32),
            pltpu.VMEM((8, _CH), jnp.int32),
            pltpu.VMEM_SHARED((np_rows, h), jnp.float32),
            pltpu.SemaphoreType.DMA,
            pltpu.SemaphoreType.DMA,
            pltpu.SemaphoreType.DMA,
        ],
        compiler_params=pltpu.CompilerParams(use_tc_tiling_on_sc=False),
    )
    def seg_sum(*refs):
        it = iter(refs)
        y_hbm = next(it)
        e_hbm = next(it)
        if use_gather_init:
            i0_hbm, ii_hbm = None, next(it)
        else:
            i0_hbm, ii_hbm = next(it), None
        z_hbm = next(it)
        out0_hbm = next(it)
        out1_hbm = next(it)
        (ev, rows0_v, rows1_v, tail_v, iidx_v, acc_sh,
         sem0, sem1, sem2) = list(it)

        c = lax.axis_index("c")
        s = lax.axis_index("s")
        w = c * _NS + s
        pltpu.sync_copy(e_hbm.at[pl.ds(main * w, main)],
                        ev.at[pl.ds(0, main)])

        @pl.when(w < ntail)
        def _():
            pltpu.sync_copy(e_hbm.at[pl.ds(main * _NW + w, 1)],
                            ev.at[pl.ds(main, 1)])

        if scale_src:
            for i in range(main):
                for j in range(_CH // 16):
                    sl = (i, 0, pl.ds(16 * j, 16))
                    ev[sl] = ev[sl] * 4

            @pl.when(w < ntail)
            def _():
                for j in range(_CH // 16):
                    sl = (main, 0, pl.ds(16 * j, 16))
                    ev[sl] = ev[sl] * 4

        def src_at(ch):
            return ev.at[ch, 0]

        def dst_at(ch):
            return ev.at[ch, 1]

        def prefetch(g, buf, sem):
            for k in range(grp):
                pltpu.async_copy(y_hbm.at[src_at(g * grp + k)],
                                 buf.at[k], sem)

        prefetch(0, rows0_v, sem0)

        @pl.when(w < ntail)
        def _():
            pltpu.async_copy(y_hbm.at[src_at(main)], tail_v, sem2)

        @pl.when(c == 0)
        def _():
            if use_gather_init:
                pltpu.sync_copy(ii_hbm.at[pl.ds(s * nir, nir)],
                                iidx_v.at[pl.ds(0, nir)])
                for k in range(nir):
                    pltpu.async_copy(y_hbm.at[iidx_v.at[k]],
                                     rows1_v.at[k % grp], sem1)
                    if k % grp == grp - 1 or k == nir - 1:
                        for k2 in range(k - k % grp, k + 1):
                            pltpu.make_async_copy(
                                y_hbm.at[iidx_v.at[k2]],
                                rows1_v.at[k2 % grp], sem1).wait()
                            pltpu.sync_copy(
                                rows1_v.at[k2 % grp],
                                acc_sh.at[pl.ds(s * ipr + k2 * _CH, _CH)])
            else:
                pltpu.sync_copy(i0_hbm.at[pl.ds(s * ipr, ipr)],
                                acc_sh.at[pl.ds(s * ipr, ipr)])

        @pl.when(c == 1)
        def _():
            pltpu.sync_copy(z_hbm.at[pl.ds(s * rpz, rpz)],
                            acc_sh.at[pl.ds(s * rpz, rpz)])

        plsc.subcore_barrier()

        def gathers(g, buf, sem):
            for k in range(grp):
                pltpu.async_copy(y_hbm.at[src_at(g * grp + k)],
                                 buf.at[k], sem)

        def drains(g, buf, sem):
            for k in range(grp):
                pltpu.make_async_copy(y_hbm.at[src_at(g * grp + k)],
                                      buf.at[k], sem).wait()

        def scatters(g, buf):
            for k in range(grp):
                pltpu.sync_copy(buf.at[k], acc_sh.at[dst_at(g * grp + k)],
                                add=True)

        def body(i, carry):
            g0 = 2 * i
            g1 = g0 + 1
            gathers(g1, rows1_v, sem1)
            drains(g0, rows0_v, sem0)
            scatters(g0, rows0_v)

            @pl.when(g1 + 1 < ngr)
            def _():
                gathers(g1 + 1, rows0_v, sem0)

            drains(g1, rows1_v, sem1)
            scatters(g1, rows1_v)
            return carry

        lax.fori_loop(0, ngr // 2, body, 0)

        @pl.when(w < ntail)
        def _():
            pltpu.make_async_copy(y_hbm.at[src_at(main)],
                                  tail_v, sem2).wait()
            pltpu.sync_copy(tail_v, acc_sh.at[dst_at(main)], add=True)

        plsc.subcore_barrier()
        @pl.when(c == 0)
        def _():
            pltpu.sync_copy(acc_sh.at[pl.ds(s * rpz, rpz)],
                            out0_hbm.at[pl.ds(s * rpz, rpz)])

        @pl.when(c == 1)
        def _():
            pltpu.sync_copy(acc_sh.at[pl.ds(s * rpz, rpz)],
                            out1_hbm.at[pl.ds(s * rpz, rpz)])

    return seg_sum(*in_types)


def _gin_in_tc(x, w1, eps):
    n, d = x.shape
    h = w1.shape[1]

    def body(x_ref, w_ref, e_ref, o_ref):
        y = jnp.dot(x_ref[...], w_ref[...],
                    preferred_element_type=jnp.float32)
        o_ref[...] = jnp.concatenate(
            [y, (1.0 + e_ref[0, 0]) * y, jnp.zeros((n, 2 * h), jnp.float32)],
            axis=1)

    return pl.pallas_call(
        body,
        out_shape=jax.ShapeDtypeStruct((n, 4 * h), jnp.float32),
    )(x, w1, eps)


def _gin_mid_tc(p0, p1, b1, w2, b2, w3, eps, nl, nlp):
    def body(p0_ref, p1_ref, b1_ref, w2_ref, b2_ref, w3_ref, e_ref,
             o1_ref, o2_ref):
        t = p0_ref[:nl] + p1_ref[:nl] + b1_ref[...]
        u = jnp.dot(jnp.maximum(t, 0.0), w2_ref[...],
                    preferred_element_type=jnp.float32) + b2_ref[...]
        y2 = jnp.dot(jnp.maximum(u, 0.0), w3_ref[...],
                     preferred_element_type=jnp.float32)
        o1_ref[pl.ds(0, nl)] = y2
        o2_ref[pl.ds(0, nl)] = (1.0 + e_ref[0, 0]) * y2

    return pl.pallas_call(
        body,
        out_shape=[jax.ShapeDtypeStruct((nlp, 128), jnp.float32),
                   jax.ShapeDtypeStruct((nlp, 128), jnp.float32)],
    )(p0, p1, b1, w2, b2, w3, eps)


def _gin_out_tc(q0, q1, b3, w4, b4, nl, nlp, h):
    def body(q0_ref, q1_ref, b3_ref, w4_ref, b4_ref, o_ref):
        t = q0_ref[:nl] + q1_ref[:nl] + b3_ref[...]
        v = jnp.dot(jnp.maximum(t, 0.0), w4_ref[...],
                    preferred_element_type=jnp.float32) + b4_ref[...]
        parts = []
        for j in range(4):
            vj = v[:, j * h:(j + 1) * h]
            mj = jnp.max(vj, axis=1, keepdims=True)
            sj = jnp.sum(jnp.exp(vj - mj), axis=1, keepdims=True)
            parts.append(vj - mj - jnp.log(sj))
        o_ref[pl.ds(0, nl)] = jnp.concatenate(parts, axis=1)

    return pl.pallas_call(
        body,
        out_shape=jax.ShapeDtypeStruct((nlp, 128), jnp.float32),
    )(q0, q1, b3, w4, b4)


def kernel(x, edge_index, W1, b1, W2, b2, eps1, W3, b3, W4, b4, eps2):
    n, d = x.shape
    h = W1.shape[1]
    e = edge_index.shape[1]
    nl = n * h // 128
    nlp = -(-nl // 8) * 8
    nrows = e // _CH

    np_rows = -(-n // (_NS * _CH)) * _NS * _CH
    npl = np_rows * h // 128

    edges = jnp.transpose(edge_index.reshape(2, nrows, _CH), (1, 0, 2))
    iidx = jnp.minimum(
        4 * jnp.arange(np_rows, dtype=jnp.int32) + 1, 4 * n - 3
    ).reshape(np_rows // _CH, _CH)
    zeros = jnp.zeros((np_rows, h), jnp.float32)
    e1 = jnp.reshape(eps1, (1, 1)).astype(jnp.float32)
    e2 = jnp.reshape(eps2, (1, 1)).astype(jnp.float32)

    eye4 = jnp.eye(4, dtype=jnp.float32)
    W2b = jnp.kron(eye4, W2)
    W3b = jnp.kron(eye4, W3)
    W4b = jnp.kron(eye4, W4)
    b1t = jnp.tile(b1, 4).reshape(1, 128)
    b2t = jnp.tile(b2, 4).reshape(1, 128)
    b3t = jnp.tile(b3, 4).reshape(1, 128)
    b4t = jnp.tile(b4, 4).reshape(1, 128)

    o1 = _gin_in_tc(x, W1, e1)
    p0, p1 = _segment_sum_sc(o1.reshape(4 * n, h), edges, True,
                             None, iidx, zeros, h)
    y2, y2s = _gin_mid_tc(p0.reshape(npl, 128), p1.reshape(npl, 128),
                          b1t, W2b, b2t, W3b, e2, nl, nlp)
    q0, q1 = _segment_sum_sc(y2.reshape(nlp * 4, h), edges, False,
                             y2s.reshape(nlp * 4, h), None, zeros, h)
    out = _gin_out_tc(q0.reshape(npl, 128), q1.reshape(npl, 128),
                      b3t, W4b, b4t, nl, nlp, h)
    return out.reshape(nlp * 4, h)[:n]
